# baseline (speedup 1.0000x reference)
import functools

import jax
import jax.numpy as jnp
from jax import lax
from jax.experimental import pallas as pl
from jax.experimental.pallas import tpu as pltpu
from jax.experimental.pallas import tpu_sc as plsc

_K = 8
_NW = 32
_IDX_CHUNK = 128
_BLOCK_B = 1024


def _sc_gather(table, idx, rows_per_w):
    D = table.shape[1]
    R = idx.shape[0]
    n_chunks = rows_per_w // _IDX_CHUNK
    assert n_chunks % 2 == 0
    mesh = plsc.VectorSubcoreMesh(core_axis_name="c", subcore_axis_name="s")

    @functools.partial(
        pl.kernel,
        out_type=jax.ShapeDtypeStruct((R, D), table.dtype),
        mesh=mesh,
        scratch_types=[
            pltpu.VMEM((_IDX_CHUNK,), jnp.int32),
            pltpu.VMEM((_IDX_CHUNK,), jnp.int32),
            pltpu.VMEM((_IDX_CHUNK, D), table.dtype),
            pltpu.VMEM((_IDX_CHUNK, D), table.dtype),
            pltpu.SemaphoreType.DMA,
            pltpu.SemaphoreType.DMA,
        ],
    )
    def gk(table_hbm, idx_hbm, out_hbm, idx0, idx1, rows0, rows1, sem0, sem1):
        wid = lax.axis_index("s") * 2 + lax.axis_index("c")
        base = wid * rows_per_w

        pltpu.sync_copy(idx_hbm.at[pl.ds(base, _IDX_CHUNK)], idx0)
        pltpu.async_copy(table_hbm.at[idx0], rows0, sem0)

        def body(j, carry):
            o0 = base + (2 * j) * _IDX_CHUNK
            o1 = o0 + _IDX_CHUNK
            pltpu.sync_copy(idx_hbm.at[pl.ds(o1, _IDX_CHUNK)], idx1)
            pltpu.async_copy(table_hbm.at[idx1], rows1, sem1)
            pltpu.make_async_copy(table_hbm.at[idx0], rows0, sem0).wait()
            pltpu.sync_copy(rows0, out_hbm.at[pl.ds(o0, _IDX_CHUNK)])

            @pl.when(2 * j + 2 < n_chunks)
            def _():
                o2 = o1 + _IDX_CHUNK
                pltpu.sync_copy(idx_hbm.at[pl.ds(o2, _IDX_CHUNK)], idx0)
                pltpu.async_copy(table_hbm.at[idx0], rows0, sem0)

            pltpu.make_async_copy(table_hbm.at[idx1], rows1, sem1).wait()
            pltpu.sync_copy(rows1, out_hbm.at[pl.ds(o1, _IDX_CHUNK)])
            return carry

        lax.fori_loop(0, n_chunks // 2, body, 0)

    return gk(table, idx)


def _lstm_chunk(X, h, c, counts_col, wih_t, whh_t, bias, t0, nv):
    K, N_pad, D = X.shape
    H = h.shape[1]

    def body(s_ref, x_ref, h_ref, c_ref, cnt_ref, wih_ref, whh_ref, b_ref,
             ho_ref, co_ref):
        t0v = s_ref[0]
        nvv = s_ref[1]
        cnt = cnt_ref[...]
        wih = wih_ref[...]
        whh = whh_ref[...]
        b = b_ref[...]

        def step(k, hc):
            hh, cc = hc
            x = jnp.where(t0v + k < cnt, x_ref[k], 0.0)
            g = jnp.dot(x.astype(jnp.bfloat16), wih,
                        preferred_element_type=jnp.float32)
            g = g + jnp.dot(hh.astype(jnp.bfloat16), whh,
                            preferred_element_type=jnp.float32)
            g = g + b
            s1 = 0.5 * jnp.tanh(0.5 * g[:, :2 * H]) + 0.5
            gi = s1[:, :H]
            gf = s1[:, H:]
            gg = jnp.tanh(g[:, 2 * H:3 * H])
            go = 0.5 * jnp.tanh(0.5 * g[:, 3 * H:]) + 0.5
            c2 = gf * cc + gi * gg
            h2 = go * jnp.tanh(c2)
            return (h2, c2)

        hh, cc = lax.fori_loop(
            0, jnp.clip(nvv, 0, K), step, (h_ref[...], c_ref[...]))
        ho_ref[...] = hh
        co_ref[...] = cc

    s_arr = jnp.stack([t0, nv]).astype(jnp.int32)
    return pl.pallas_call(
        body,
        grid=(N_pad // _BLOCK_B,),
        in_specs=[
            pl.BlockSpec(memory_space=pltpu.SMEM),
            pl.BlockSpec((K, _BLOCK_B, D), lambda i: (0, i, 0)),
            pl.BlockSpec((_BLOCK_B, H), lambda i: (i, 0)),
            pl.BlockSpec((_BLOCK_B, H), lambda i: (i, 0)),
            pl.BlockSpec((_BLOCK_B, 1), lambda i: (i, 0)),
            pl.BlockSpec((D, 4 * H), lambda i: (0, 0)),
            pl.BlockSpec((H, 4 * H), lambda i: (0, 0)),
            pl.BlockSpec((1, 4 * H), lambda i: (0, 0)),
        ],
        out_specs=[
            pl.BlockSpec((_BLOCK_B, H), lambda i: (i, 0)),
            pl.BlockSpec((_BLOCK_B, H), lambda i: (i, 0)),
        ],
        out_shape=[
            jax.ShapeDtypeStruct((N_pad, H), jnp.float32),
            jax.ShapeDtypeStruct((N_pad, H), jnp.float32),
        ],
    )(s_arr, X, h, c, counts_col, wih_t, whh_t, bias)


def _final_linear(h, x, wl_t, wr_t, b):
    N_pad, H = h.shape
    D = x.shape[1]

    def body(h_ref, x_ref, wl_ref, wr_ref, b_ref, o_ref):
        o = jnp.dot(h_ref[...], wl_ref[...], preferred_element_type=jnp.float32)
        o = o + jnp.dot(x_ref[...], wr_ref[...], preferred_element_type=jnp.float32)
        o = o + b_ref[...]
        o_ref[...] = jnp.maximum(o, 0.0)

    return pl.pallas_call(
        body,
        grid=(N_pad // _BLOCK_B,),
        in_specs=[
            pl.BlockSpec((_BLOCK_B, H), lambda i: (i, 0)),
            pl.BlockSpec((_BLOCK_B, D), lambda i: (i, 0)),
            pl.BlockSpec((H, H), lambda i: (0, 0)),
            pl.BlockSpec((D, H), lambda i: (0, 0)),
            pl.BlockSpec((1, H), lambda i: (0, 0)),
        ],
        out_specs=pl.BlockSpec((_BLOCK_B, H), lambda i: (i, 0)),
        out_shape=jax.ShapeDtypeStruct((N_pad, H), jnp.float32),
    )(h, x, wl_t, wr_t, b)


def kernel(node_feats, edge_index, W_ih, W_hh, b_ih, b_hh, W_l, b_l, W_r):
    N, D = node_feats.shape
    H = W_hh.shape[1]
    E = edge_index.shape[1]

    src = edge_index[0]
    dst = edge_index[1]
    order = jnp.argsort(dst)
    src_s = src[order].astype(jnp.int32)
    counts = jnp.bincount(dst, length=N).astype(jnp.int32)
    ptr = (jnp.cumsum(counts) - counts).astype(jnp.int32)
    max_deg = jnp.max(counts)

    N_pad = ((N + _BLOCK_B - 1) // _BLOCK_B) * _BLOCK_B
    rows_per_w = _K * N_pad // _NW

    counts_p = jnp.pad(counts, (0, N_pad - N))
    counts_col = counts_p[:, None]
    ptr_p = jnp.pad(ptr, (0, N_pad - N))

    wih_t = W_ih.T.astype(jnp.bfloat16)
    whh_t = W_hh.T.astype(jnp.bfloat16)
    bias = (b_ih + b_hh)[None, :]

    ts_base = jnp.arange(_K, dtype=jnp.int32)

    def gather_chunk(m):
        pos = ptr_p[None, :] + (m * _K + ts_base)[:, None]
        sidx = jnp.take(src_s, pos, mode="clip")
        return _sc_gather(node_feats, sidx.reshape(-1), rows_per_w)

    def lstm_step(h, c, X, m):
        t0 = m * _K
        nv = jnp.minimum(_K, max_deg - t0)
        return _lstm_chunk(X.reshape(_K, N_pad, D), h, c, counts_col,
                           wih_t, whh_t, bias, t0, nv)

    def chunk_body(j, carry):
        h, c, XA, XB = carry
        m = 2 * j
        h, c = lstm_step(h, c, XA, m)
        XA2 = gather_chunk(m + 2)
        h, c = lstm_step(h, c, XB, m + 1)
        XB2 = gather_chunk(m + 3)
        return (h, c, XA2, XB2)

    nc2 = (max_deg + 2 * _K - 1) // (2 * _K)
    h0 = jnp.zeros((N_pad, H), jnp.float32)
    c0 = jnp.zeros((N_pad, H), jnp.float32)
    h, c, XA, XB = lax.fori_loop(
        0, nc2 - 1, chunk_body, (h0, c0, gather_chunk(0), gather_chunk(1)))
    m_last = 2 * (nc2 - 1)
    h, c = lstm_step(h, c, XA, m_last)
    h, _ = lstm_step(h, c, XB, m_last + 1)

    x_p = jnp.concatenate(
        [node_feats, jnp.zeros((N_pad - N, D), node_feats.dtype)], axis=0)
    out = _final_linear(h, x_p, W_l.T, W_r.T, b_l[None, :])
    return out[:N]


# speedup vs baseline: 1.0018x; 1.0018x over previous
import functools

import jax
import jax.numpy as jnp
from jax import lax
from jax.experimental import pallas as pl
from jax.experimental.pallas import tpu as pltpu
from jax.experimental.pallas import tpu_sc as plsc

_K = 8
_NW = 32
_IDX_CHUNK = 128
_BLOCK_B = 1024


def _sc_gather(table, idx, rows_per_w):
    D = table.shape[1]
    R = idx.shape[0]
    n_chunks = rows_per_w // _IDX_CHUNK
    assert n_chunks % 2 == 0
    mesh = plsc.VectorSubcoreMesh(core_axis_name="c", subcore_axis_name="s")

    @functools.partial(
        pl.kernel,
        out_type=jax.ShapeDtypeStruct((R, D), table.dtype),
        mesh=mesh,
        scratch_types=[
            pltpu.VMEM((_IDX_CHUNK,), jnp.int32),
            pltpu.VMEM((_IDX_CHUNK,), jnp.int32),
            pltpu.VMEM((_IDX_CHUNK, D), table.dtype),
            pltpu.VMEM((_IDX_CHUNK, D), table.dtype),
            pltpu.SemaphoreType.DMA,
            pltpu.SemaphoreType.DMA,
        ],
    )
    def gk(table_hbm, idx_hbm, out_hbm, idx0, idx1, rows0, rows1, sem0, sem1):
        wid = lax.axis_index("s") * 2 + lax.axis_index("c")
        base = wid * rows_per_w

        pltpu.sync_copy(idx_hbm.at[pl.ds(base, _IDX_CHUNK)], idx0)
        pltpu.async_copy(table_hbm.at[idx0], rows0, sem0)

        def body(j, carry):
            o0 = base + (2 * j) * _IDX_CHUNK
            o1 = o0 + _IDX_CHUNK
            pltpu.sync_copy(idx_hbm.at[pl.ds(o1, _IDX_CHUNK)], idx1)
            pltpu.async_copy(table_hbm.at[idx1], rows1, sem1)
            pltpu.make_async_copy(table_hbm.at[idx0], rows0, sem0).wait()
            pltpu.sync_copy(rows0, out_hbm.at[pl.ds(o0, _IDX_CHUNK)])

            @pl.when(2 * j + 2 < n_chunks)
            def _():
                o2 = o1 + _IDX_CHUNK
                pltpu.sync_copy(idx_hbm.at[pl.ds(o2, _IDX_CHUNK)], idx0)
                pltpu.async_copy(table_hbm.at[idx0], rows0, sem0)

            pltpu.make_async_copy(table_hbm.at[idx1], rows1, sem1).wait()
            pltpu.sync_copy(rows1, out_hbm.at[pl.ds(o1, _IDX_CHUNK)])
            return carry

        lax.fori_loop(0, n_chunks // 2, body, 0)

    return gk(table, idx)


def _lstm_chunk(X, h, c, counts_col, wih_t, whh_t, bias, t0, nv):
    K, N_pad, D = X.shape
    H = h.shape[1]

    def body(s_ref, x_ref, h_ref, c_ref, cnt_ref, wih_ref, whh_ref, b_ref,
             ho_ref, co_ref):
        t0v = s_ref[0]
        nvv = s_ref[1]
        hh = h_ref[...]
        cc = c_ref[...]
        cnt = cnt_ref[...]
        whh = whh_ref[...]
        b = b_ref[...]
        xw = jnp.dot(x_ref[...].reshape(K * _BLOCK_B, D).astype(jnp.bfloat16),
                     wih_ref[...], preferred_element_type=jnp.float32)
        for k in range(K):
            xwk = xw[k * _BLOCK_B:(k + 1) * _BLOCK_B]
            g = jnp.where(t0v + k < cnt, xwk, 0.0)
            g = g + jnp.dot(hh.astype(whh.dtype), whh,
                            preferred_element_type=jnp.float32)
            g = g + b
            s1 = 0.5 * jnp.tanh(0.5 * g[:, :2 * H]) + 0.5
            gi = s1[:, :H]
            gf = s1[:, H:]
            gg = jnp.tanh(g[:, 2 * H:3 * H])
            go = 0.5 * jnp.tanh(0.5 * g[:, 3 * H:]) + 0.5
            c2 = gf * cc + gi * gg
            h2 = go * jnp.tanh(c2)
            keep = k < nvv
            hh = jnp.where(keep, h2, hh)
            cc = jnp.where(keep, c2, cc)
        ho_ref[...] = hh
        co_ref[...] = cc

    s_arr = jnp.stack([t0, nv]).astype(jnp.int32)
    return pl.pallas_call(
        body,
        grid=(N_pad // _BLOCK_B,),
        in_specs=[
            pl.BlockSpec(memory_space=pltpu.SMEM),
            pl.BlockSpec((K, _BLOCK_B, D), lambda i: (0, i, 0)),
            pl.BlockSpec((_BLOCK_B, H), lambda i: (i, 0)),
            pl.BlockSpec((_BLOCK_B, H), lambda i: (i, 0)),
            pl.BlockSpec((_BLOCK_B, 1), lambda i: (i, 0)),
            pl.BlockSpec((D, 4 * H), lambda i: (0, 0)),
            pl.BlockSpec((H, 4 * H), lambda i: (0, 0)),
            pl.BlockSpec((1, 4 * H), lambda i: (0, 0)),
        ],
        out_specs=[
            pl.BlockSpec((_BLOCK_B, H), lambda i: (i, 0)),
            pl.BlockSpec((_BLOCK_B, H), lambda i: (i, 0)),
        ],
        out_shape=[
            jax.ShapeDtypeStruct((N_pad, H), jnp.float32),
            jax.ShapeDtypeStruct((N_pad, H), jnp.float32),
        ],
    )(s_arr, X, h, c, counts_col, wih_t, whh_t, bias)


def _final_linear(h, x, wl_t, wr_t, b):
    N_pad, H = h.shape
    D = x.shape[1]

    def body(h_ref, x_ref, wl_ref, wr_ref, b_ref, o_ref):
        o = jnp.dot(h_ref[...], wl_ref[...], preferred_element_type=jnp.float32)
        o = o + jnp.dot(x_ref[...], wr_ref[...], preferred_element_type=jnp.float32)
        o = o + b_ref[...]
        o_ref[...] = jnp.maximum(o, 0.0)

    return pl.pallas_call(
        body,
        grid=(N_pad // _BLOCK_B,),
        in_specs=[
            pl.BlockSpec((_BLOCK_B, H), lambda i: (i, 0)),
            pl.BlockSpec((_BLOCK_B, D), lambda i: (i, 0)),
            pl.BlockSpec((H, H), lambda i: (0, 0)),
            pl.BlockSpec((D, H), lambda i: (0, 0)),
            pl.BlockSpec((1, H), lambda i: (0, 0)),
        ],
        out_specs=pl.BlockSpec((_BLOCK_B, H), lambda i: (i, 0)),
        out_shape=jax.ShapeDtypeStruct((N_pad, H), jnp.float32),
    )(h, x, wl_t, wr_t, b)


def kernel(node_feats, edge_index, W_ih, W_hh, b_ih, b_hh, W_l, b_l, W_r):
    N, D = node_feats.shape
    H = W_hh.shape[1]
    E = edge_index.shape[1]

    src = edge_index[0]
    dst = edge_index[1]
    order = jnp.argsort(dst)
    src_s = src[order].astype(jnp.int32)
    counts = jnp.bincount(dst, length=N).astype(jnp.int32)
    ptr = (jnp.cumsum(counts) - counts).astype(jnp.int32)
    max_deg = jnp.max(counts)

    N_pad = ((N + _BLOCK_B - 1) // _BLOCK_B) * _BLOCK_B
    rows_per_w = _K * N_pad // _NW

    counts_p = jnp.pad(counts, (0, N_pad - N))
    counts_col = counts_p[:, None]
    ptr_p = jnp.pad(ptr, (0, N_pad - N))

    wih_t = W_ih.T.astype(jnp.bfloat16)
    whh_t = W_hh.T.astype(jnp.bfloat16)
    bias = (b_ih + b_hh)[None, :]

    ts_base = jnp.arange(_K, dtype=jnp.int32)

    def gather_chunk(m):
        pos = ptr_p[None, :] + (m * _K + ts_base)[:, None]
        sidx = jnp.take(src_s, pos, mode="clip")
        return _sc_gather(node_feats, sidx.reshape(-1), rows_per_w)

    def lstm_step(h, c, X, m):
        t0 = m * _K
        nv = jnp.minimum(_K, max_deg - t0)
        return _lstm_chunk(X.reshape(_K, N_pad, D), h, c, counts_col,
                           wih_t, whh_t, bias, t0, nv)

    def chunk_body(j, carry):
        h, c, XA, XB = carry
        m = 2 * j
        h, c = lstm_step(h, c, XA, m)
        XA2 = gather_chunk(m + 2)
        h, c = lstm_step(h, c, XB, m + 1)
        XB2 = gather_chunk(m + 3)
        return (h, c, XA2, XB2)

    nc2 = (max_deg + 2 * _K - 1) // (2 * _K)
    h0 = jnp.zeros((N_pad, H), jnp.float32)
    c0 = jnp.zeros((N_pad, H), jnp.float32)
    h, c, XA, XB = lax.fori_loop(
        0, nc2 - 1, chunk_body, (h0, c0, gather_chunk(0), gather_chunk(1)))
    m_last = 2 * (nc2 - 1)
    h, c = lstm_step(h, c, XA, m_last)
    h, _ = lstm_step(h, c, XB, m_last + 1)

    x_p = jnp.concatenate(
        [node_feats, jnp.zeros((N_pad - N, D), node_feats.dtype)], axis=0)
    out = _final_linear(h, x_p, W_l.T, W_r.T, b_l[None, :])
    return out[:N]
